# Initial kernel scaffold; baseline (speedup 1.0000x reference)
import jax
import jax.numpy as jnp
from jax.experimental import pallas as pl


def kernel(hidden_states, router_weight, gate_up_proj, down_proj):
    raise NotImplementedError("write your pallas kernel here")


# speedup vs baseline: 1.1982x; 1.1982x over previous
import functools

import jax
import jax.numpy as jnp
from jax.experimental import pallas as pl
from jax.experimental.pallas import tpu as pltpu

HIDDEN = 2048
FFN = 2048
NUM_EXPERTS = 8
TOP_K = 2
JITTER = 0.01
SEQ = 2048
LANES = 128
NEG = -1e30


def _router_body(x_ref, wrt_ref, we_ref):
    lane = jax.lax.broadcasted_iota(jnp.int32, (SEQ, LANES), 1)
    scores = jax.lax.dot_general(
        x_ref[...], wrt_ref[...], (((1,), (0,)), ((), ())),
        precision=jax.lax.Precision.DEFAULT)
    s = jnp.where(lane < NUM_EXPERTS, scores, NEG)

    max1 = jnp.max(s, axis=1, keepdims=True)
    ind1 = jnp.min(jnp.where(s == max1, lane, LANES), axis=1, keepdims=True)
    factor1 = jnp.maximum(jnp.abs(s), max1)
    mask1 = ((max1 - s) / factor1) > (2.0 * JITTER)
    oh1 = lane == ind1
    masked = jnp.where(oh1, NEG, s)
    max2 = jnp.max(masked, axis=1, keepdims=True)
    ind2 = jnp.min(jnp.where(masked == max2, lane, LANES), axis=1, keepdims=True)
    factor2 = jnp.maximum(jnp.abs(s), max2)
    mask2 = ((max2 - s) / factor2) > (2.0 * JITTER)

    def _softmax(a):
        z = a - jnp.max(a, axis=1, keepdims=True)
        e = jnp.exp(z)
        return e / jnp.sum(e, axis=1, keepdims=True)

    mg1 = _softmax(jnp.where(mask1, NEG, s))
    m1 = jnp.sum(jnp.where(oh1, mg1, 0.0), axis=1, keepdims=True)
    mg2 = _softmax(jnp.where(mask2, NEG, masked))
    m2 = jnp.sum(jnp.where(lane == ind2, mg2, 0.0), axis=1, keepdims=True)

    we = jnp.where(oh1, m1, 0.0) + jnp.where(lane == ind2, m2, 0.0)
    we_ref[...] = we


def _router(x2d, wrt):
    return pl.pallas_call(
        _router_body,
        out_shape=jax.ShapeDtypeStruct((SEQ, LANES), jnp.float32),
    )(x2d, wrt)


def _dense_ffn_body(x_ref, wg_ref, wu_ref, dwn_ref, we_ref, out_ref):
    e = pl.program_id(1)
    f = pl.program_id(2)

    @pl.when(jnp.logical_and(e == 0, f == 0))
    def _():
        out_ref[...] = jnp.zeros_like(out_ref)

    bt = x_ref.shape[0]
    lane = jax.lax.broadcasted_iota(jnp.int32, (bt, LANES), 1)
    wcol = jnp.sum(jnp.where(lane == e, we_ref[...], 0.0), axis=1, keepdims=True)
    xb = x_ref[...].astype(jnp.bfloat16)
    wg = wg_ref[0].astype(jnp.bfloat16)
    wu = wu_ref[0].astype(jnp.bfloat16)
    g = jnp.dot(xb, wg, preferred_element_type=jnp.float32)
    u = jnp.dot(xb, wu, preferred_element_type=jnp.float32)
    act = (g * (1.0 / (1.0 + jnp.exp(-g))) * u).astype(jnp.bfloat16)
    wd = dwn_ref[0].astype(jnp.bfloat16)
    out_ref[...] += jnp.dot(act, wd, preferred_element_type=jnp.float32) * wcol


def _dense_ffn(x2d, gate_up, down, we):
    BT = 512
    FT = 512
    NF = FFN // FT
    grid = (SEQ // BT, NUM_EXPERTS, NF)
    return pl.pallas_call(
        _dense_ffn_body,
        grid=grid,
        in_specs=[
            pl.BlockSpec((BT, HIDDEN), lambda i, e, f: (i, 0)),
            pl.BlockSpec((1, HIDDEN, FT), lambda i, e, f: (e, 0, f)),
            pl.BlockSpec((1, HIDDEN, FT), lambda i, e, f: (e, 0, NF + f)),
            pl.BlockSpec((1, FT, HIDDEN), lambda i, e, f: (e, f, 0)),
            pl.BlockSpec((BT, LANES), lambda i, e, f: (i, 0)),
        ],
        out_specs=pl.BlockSpec((BT, HIDDEN), lambda i, e, f: (i, 0)),
        out_shape=jax.ShapeDtypeStruct((SEQ, HIDDEN), jnp.float32),
    )(x2d, gate_up, gate_up, down, we)


@jax.jit
def kernel(hidden_states, router_weight, gate_up_proj, down_proj):
    B, S, H = hidden_states.shape
    x2d = hidden_states.reshape(-1, H)
    wrt = jnp.zeros((H, LANES), jnp.float32).at[:, :NUM_EXPERTS].set(router_weight.T)
    we = _router(x2d, wrt)
    out = _dense_ffn(x2d, gate_up_proj, down_proj, we)
    return out.reshape(B, S, H)
